# Initial kernel scaffold; baseline (speedup 1.0000x reference)
import jax
import jax.numpy as jnp
from jax.experimental import pallas as pl


def kernel(indices, table):
    raise NotImplementedError("write your pallas kernel here")


# speedup vs baseline: 4.1691x; 4.1691x over previous
import functools

import jax
import jax.numpy as jnp
from jax import lax
from jax.experimental import pallas as pl
from jax.experimental.pallas import tpu as pltpu
from jax.experimental.pallas import tpu_sc as plsc

NC = 2
NS = 16
NW = NC * NS
SUB = 128
NSUB = 8
CHUNK = SUB * NSUB


@functools.lru_cache(maxsize=None)
def _make(nchunk, vocab, dim):
    mesh = plsc.VectorSubcoreMesh(core_axis_name="c", subcore_axis_name="s")

    @functools.partial(
        pl.kernel,
        mesh=mesh,
        out_type=jax.ShapeDtypeStruct((NW, nchunk, CHUNK, dim), jnp.float32),
        compiler_params=pltpu.CompilerParams(use_tc_tiling_on_sc=False),
        scratch_types=[
            pltpu.VMEM((NSUB, SUB), jnp.int32),
            pltpu.VMEM((CHUNK, dim), jnp.float32),
            pltpu.SemaphoreType.DMA,
        ],
    )
    def k(idx_hbm, table_hbm, out_hbm, idx_v, rows_v, sem):
        wid = lax.axis_index("s") * NC + lax.axis_index("c")

        def body(c, carry):
            pltpu.sync_copy(idx_hbm.at[wid, c], idx_v)
            copies = [
                pltpu.async_copy(
                    table_hbm.at[idx_v.at[j]],
                    rows_v.at[pl.ds(j * SUB, SUB)],
                    sem,
                )
                for j in range(NSUB)
            ]
            for cp in copies:
                cp.wait()
            pltpu.sync_copy(rows_v, out_hbm.at[wid, c])
            return carry

        lax.fori_loop(0, nchunk, body, 0)

    return k


def kernel(indices, table):
    n, m = indices.shape
    vocab, dim = table.shape
    b = n * m
    nchunk = b // (NW * CHUNK)
    idx = indices.astype(jnp.int32).reshape(NW, nchunk, NSUB, SUB)
    out = _make(nchunk, vocab, dim)(idx, table)
    return out.reshape(n, m, dim)


# speedup vs baseline: 6.8789x; 1.6500x over previous
import functools

import jax
import jax.numpy as jnp
from jax import lax
from jax.experimental import pallas as pl
from jax.experimental.pallas import tpu as pltpu
from jax.experimental.pallas import tpu_sc as plsc

NC = 2
NS = 16
NW = NC * NS
SUB = 128
NSUB = 8
CHUNK = SUB * NSUB


@functools.lru_cache(maxsize=None)
def _make(nchunk, vocab, dim):
    mesh = plsc.VectorSubcoreMesh(core_axis_name="c", subcore_axis_name="s")
    assert nchunk % 2 == 0

    @functools.partial(
        pl.kernel,
        mesh=mesh,
        out_type=jax.ShapeDtypeStruct((NW, nchunk, CHUNK, dim), jnp.float32),
        compiler_params=pltpu.CompilerParams(use_tc_tiling_on_sc=False),
        scratch_types=[
            pltpu.VMEM((2, NSUB, SUB), jnp.int32),
            pltpu.VMEM((2, CHUNK, dim), jnp.float32),
            pltpu.VMEM_SHARED((vocab, dim), jnp.float32),
            pltpu.SemaphoreType.DMA,
            pltpu.SemaphoreType.DMA,
            pltpu.SemaphoreType.DMA,
            pltpu.SemaphoreType.DMA,
        ],
    )
    def k(idx_hbm, table_hbm, out_hbm, idx_v, rows_v, table_sh,
          gsem0, gsem1, osem0, osem1):
        wid = lax.axis_index("s") * NC + lax.axis_index("c")
        gsems = (gsem0, gsem1)
        osems = (osem0, osem1)

        @pl.when(lax.axis_index("s") == 0)
        def _():
            pltpu.sync_copy(table_hbm, table_sh)

        plsc.subcore_barrier()

        def fire(c, b):
            pltpu.sync_copy(idx_hbm.at[wid, c], idx_v.at[b])
            for j in range(NSUB):
                pltpu.async_copy(
                    table_sh.at[idx_v.at[b, j]],
                    rows_v.at[b, pl.ds(j * SUB, SUB)],
                    gsems[b],
                )

        def wait_gathers(b):
            pltpu.make_async_copy(out_hbm.at[wid, 0], rows_v.at[b],
                                  gsems[b]).wait()

        def start_out(c, b):
            pltpu.async_copy(rows_v.at[b], out_hbm.at[wid, c], osems[b])

        def wait_out(b):
            pltpu.make_async_copy(rows_v.at[b], out_hbm.at[wid, 0],
                                  osems[b]).wait()

        fire(0, 0)
        ng = nchunk // 2

        def body(g, carry):
            c = g * 2

            @pl.when(g > 0)
            def _():
                wait_out(1)

            fire(c + 1, 1)
            wait_gathers(0)
            start_out(c, 0)
            wait_out(0)

            @pl.when(g < ng - 1)
            def _():
                fire(c + 2, 0)

            wait_gathers(1)
            start_out(c + 1, 1)
            return carry

        lax.fori_loop(0, ng, body, 0)
        wait_out(1)

    return k


def kernel(indices, table):
    n, m = indices.shape
    vocab, dim = table.shape
    b = n * m
    nchunk = b // (NW * CHUNK)
    idx = indices.astype(jnp.int32).reshape(NW, nchunk, NSUB, SUB)
    out = _make(nchunk, vocab, dim)(idx, table)
    return out.reshape(n, m, dim)
